# baseline (speedup 1.0000x reference)
import functools

import jax
import jax.numpy as jnp
from jax import lax
from jax.experimental import pallas as pl
from jax.experimental.pallas import tpu as pltpu
from jax.experimental.pallas import tpu_sc as plsc

T = 2048
N = 4096
L = 16
NV = N // L
TOPK = 6
NC = 2
NS = 16
NW = NC * NS
ROWS = T // NW
NB = 8
NQ = ROWS // NB
EPS = 1e-7
NEG = -jnp.inf


def _sortd(v):
    return plsc.sort_key_val(v, v, descending=True)[0]


def _merge16(a, b):
    return _sortd(jnp.maximum(a, jnp.flip(b)))


def _topk_stats(sorted_cands):
    m = sorted_cands[0]
    for t in sorted_cands[1:]:
        m = _merge16(m, t)
    delta = m[TOPK - 1]
    sumv = jnp.maximum(m - delta, 0.0)
    s = jnp.sum(sumv) + EPS
    return delta, s


def _sc_body(x_hbm, out_hbm, *bufs):
    xv = list(bufs[0:NB])
    wv = list(bufs[NB:2 * NB])
    si = list(bufs[2 * NB:3 * NB])
    so = list(bufs[3 * NB:4 * NB])
    wid = lax.axis_index("s") * NC + lax.axis_index("c")
    base = wid * ROWS

    for j in range(NB):
        pltpu.async_copy(x_hbm.at[base + j], xv[j], si[j])

    def p1_top3(bslot):
        def body(i, carry):
            t0, t1, t2 = carry
            v = xv[bslot][pl.ds(i * L, L)]
            n0 = jnp.maximum(t0, v); v = jnp.minimum(t0, v)
            n1 = jnp.maximum(t1, v); v = jnp.minimum(t1, v)
            n2 = jnp.maximum(t2, v)
            return (n0, n1, n2)

        neg = jnp.full((L,), NEG, jnp.float32)
        return lax.fori_loop(0, NV, body, (neg,) * 3, unroll=8)

    def p1_top6(bslot):
        def body(i, carry):
            t0, t1, t2, t3, t4, t5 = carry
            v = xv[bslot][pl.ds(i * L, L)]
            n0 = jnp.maximum(t0, v); v = jnp.minimum(t0, v)
            n1 = jnp.maximum(t1, v); v = jnp.minimum(t1, v)
            n2 = jnp.maximum(t2, v); v = jnp.minimum(t2, v)
            n3 = jnp.maximum(t3, v); v = jnp.minimum(t3, v)
            n4 = jnp.maximum(t4, v); v = jnp.minimum(t4, v)
            n5 = jnp.maximum(t5, v)
            return (n0, n1, n2, n3, n4, n5)

        neg = jnp.full((L,), NEG, jnp.float32)
        return lax.fori_loop(0, NV, body, (neg,) * 6, unroll=8)

    def quad(q, _):
        row0 = base + NB * q

        for j in range(NB):
            pltpu.make_async_copy(x_hbm.at[row0 + j], xv[j],
                                  si[j]).wait()

            sc3 = [_sortd(c) for c in p1_top3(j)]
            stats3 = _topk_stats(sc3)
            safe = sc3[2][0] <= stats3[0]

            def slow(j=j):
                return _topk_stats([_sortd(c) for c in p1_top6(j)])

            delta, s = lax.cond(safe, lambda: stats3, slow)
            inv = jnp.ones((L,), jnp.float32) / jnp.full((L,), s,
                                                         jnp.float32)

            @pl.when(q > 0)
            def _():
                pltpu.make_async_copy(wv[j], out_hbm.at[row0 + j - NB],
                                      so[j]).wait()

            @plsc.parallel_loop(0, NV, step=1, unroll=8)
            def p4(i):
                v = xv[j][pl.ds(i * L, L)]
                wv[j][pl.ds(i * L, L)] = jnp.maximum(v - delta, 0.0) * inv

            pltpu.async_copy(wv[j], out_hbm.at[row0 + j], so[j])

            @pl.when(q < NQ - 1)
            def _():
                pltpu.async_copy(x_hbm.at[row0 + j + NB], xv[j],
                                 si[j])

        return 0

    lax.fori_loop(0, NQ, quad, 0)
    for j in range(NB):
        pltpu.make_async_copy(wv[j], out_hbm.at[base + ROWS - NB + j],
                              so[j]).wait()


@jax.jit
def kernel(attn_s):
    x = attn_s.reshape(T, N)
    mesh = plsc.VectorSubcoreMesh(core_axis_name="c", subcore_axis_name="s")
    f = pl.kernel(
        _sc_body,
        out_type=jax.ShapeDtypeStruct((T, N), jnp.float32),
        mesh=mesh,
        scratch_types=(
            [pltpu.VMEM((N,), jnp.float32)] * (2 * NB)
            + [pltpu.SemaphoreType.DMA] * (2 * NB)
        ),
        compiler_params=pltpu.CompilerParams(needs_layout_passes=False),
    )
    return f(x)


# speedup vs baseline: 1.1434x; 1.1434x over previous
import functools

import jax
import jax.numpy as jnp
from jax import lax
from jax.experimental import pallas as pl
from jax.experimental.pallas import tpu as pltpu
from jax.experimental.pallas import tpu_sc as plsc

T = 2048
N = 4096
L = 16
NV = N // L
TOPK = 6
NC = 2
NS = 16
NW = NC * NS
ROWS = T // NW
NB = 4
NQ = ROWS // NB
EPS = 1e-7
NEG = -jnp.inf


def _sortd(v):
    return plsc.sort_key_val(v, v, descending=True)[0]


def _merge16(a, b):
    return _sortd(jnp.maximum(a, jnp.flip(b)))


def _topk_stats(sorted_cands):
    m = sorted_cands[0]
    for t in sorted_cands[1:]:
        m = _merge16(m, t)
    delta = m[TOPK - 1]
    sumv = jnp.maximum(m - delta, 0.0)
    s = jnp.sum(sumv) + EPS
    return delta, s


def _sc_body(x_hbm, out_hbm, *bufs):
    xv = list(bufs[0:NB])
    wv = list(bufs[NB:2 * NB])
    si = list(bufs[2 * NB:3 * NB])
    so = list(bufs[3 * NB:4 * NB])
    wid = lax.axis_index("s") * NC + lax.axis_index("c")
    base = wid * ROWS

    for j in range(NB):
        pltpu.async_copy(x_hbm.at[base + j], xv[j], si[j])

    def p1_top3(bslot):
        def body(i, carry):
            t0, t1, t2 = carry
            v = xv[bslot][pl.ds(i * L, L)]
            n0 = jnp.maximum(t0, v); v = jnp.minimum(t0, v)
            n1 = jnp.maximum(t1, v); v = jnp.minimum(t1, v)
            n2 = jnp.maximum(t2, v)
            return (n0, n1, n2)

        neg = jnp.full((L,), NEG, jnp.float32)
        return lax.fori_loop(0, NV, body, (neg,) * 3, unroll=16)

    def p1_top6(bslot):
        def body(i, carry):
            t0, t1, t2, t3, t4, t5 = carry
            v = xv[bslot][pl.ds(i * L, L)]
            n0 = jnp.maximum(t0, v); v = jnp.minimum(t0, v)
            n1 = jnp.maximum(t1, v); v = jnp.minimum(t1, v)
            n2 = jnp.maximum(t2, v); v = jnp.minimum(t2, v)
            n3 = jnp.maximum(t3, v); v = jnp.minimum(t3, v)
            n4 = jnp.maximum(t4, v); v = jnp.minimum(t4, v)
            n5 = jnp.maximum(t5, v)
            return (n0, n1, n2, n3, n4, n5)

        neg = jnp.full((L,), NEG, jnp.float32)
        return lax.fori_loop(0, NV, body, (neg,) * 6, unroll=8)

    def quad(q, _):
        row0 = base + NB * q

        for j in range(NB):
            pltpu.make_async_copy(x_hbm.at[row0 + j], xv[j],
                                  si[j]).wait()

            sc3 = [_sortd(c) for c in p1_top3(j)]
            stats3 = _topk_stats(sc3)
            safe = sc3[2][0] <= stats3[0]

            def slow(j=j):
                return _topk_stats([_sortd(c) for c in p1_top6(j)])

            delta, s = lax.cond(safe, lambda: stats3, slow)
            inv = jnp.ones((L,), jnp.float32) / jnp.full((L,), s,
                                                         jnp.float32)

            @pl.when(q > 0)
            def _():
                pltpu.make_async_copy(wv[j], out_hbm.at[row0 + j - NB],
                                      so[j]).wait()

            @plsc.parallel_loop(0, NV, step=1, unroll=8)
            def p4(i):
                v = xv[j][pl.ds(i * L, L)]
                wv[j][pl.ds(i * L, L)] = jnp.maximum(v - delta, 0.0) * inv

            pltpu.async_copy(wv[j], out_hbm.at[row0 + j], so[j])

            @pl.when(q < NQ - 1)
            def _():
                pltpu.async_copy(x_hbm.at[row0 + j + NB], xv[j],
                                 si[j])

        return 0

    lax.fori_loop(0, NQ, quad, 0)
    for j in range(NB):
        pltpu.make_async_copy(wv[j], out_hbm.at[base + ROWS - NB + j],
                              so[j]).wait()


@jax.jit
def kernel(attn_s):
    x = attn_s.reshape(T, N)
    mesh = plsc.VectorSubcoreMesh(core_axis_name="c", subcore_axis_name="s")
    f = pl.kernel(
        _sc_body,
        out_type=jax.ShapeDtypeStruct((T, N), jnp.float32),
        mesh=mesh,
        scratch_types=(
            [pltpu.VMEM((N,), jnp.float32)] * (2 * NB)
            + [pltpu.SemaphoreType.DMA] * (2 * NB)
        ),
        compiler_params=pltpu.CompilerParams(needs_layout_passes=False),
    )
    return f(x)
